# baseline (speedup 1.0000x reference)
import jax
import jax.numpy as jnp
from jax import lax
from jax.experimental import pallas as pl
from jax.experimental.pallas import tpu as pltpu
from jax.experimental.pallas import tpu_sc as plsc

B = 4
T = 8192
D = 1024
NC = 2
NS = 16
NW = NC * NS
P_PER_W = T // NW
C = 16
H = C // 2
NCH = P_PER_W // C
L = 16


def _body(x_hbm, tok_hbm, pos_hbm, out_hbm, idx_v, pos_v,
          t0, t1, t2, t3, ob0, ob1,
          g0, g1, g2, g3, w0, w1, psem):
    tok = (t0, t1, t2, t3)
    ob = (ob0, ob1)
    gsem = (g0, g1, g2, g3)
    wsem = (w0, w1)
    cid = lax.axis_index("c")
    sid = lax.axis_index("s")
    wid = sid * NC + cid
    p0 = wid * P_PER_W

    def g_issue(c, b):
        pltpu.async_copy(
            tok_hbm.at[idx_v.at[pl.ds(b * P_PER_W + c * C, C)]],
            tok[b], gsem[b])

    def g_wait(b):
        pltpu.make_async_copy(
            tok_hbm.at[idx_v.at[pl.ds(0, C)]], tok[b], gsem[b]).wait()

    def w_issue_h(c, b, h):
        pltpu.async_copy(ob[b % 2].at[pl.ds(h * H, H)],
                         out_hbm.at[b, pl.ds(p0 + c * C + h * H, H)],
                         wsem[b % 2])

    def w_wait(k):
        for _ in range(2):
            pltpu.make_async_copy(ob[k].at[pl.ds(0, H)],
                                  out_hbm.at[0, pl.ds(0, H)],
                                  wsem[k]).wait()

    def p_issue(c):
        pltpu.async_copy(pos_hbm.at[pl.ds(p0 + c * C, C)], pos_v, psem)

    def p_wait():
        pltpu.make_async_copy(pos_hbm.at[pl.ds(0, C)], pos_v, psem).wait()

    def add_pair(c, ph):
        s0, s1 = tok[2 * ph], tok[2 * ph + 1]

        def row(r, acc):
            o0_r = ob0.at[r]
            o1_r = ob1.at[r]
            s0_r = s0.at[r]
            s1_r = s1.at[r]
            p_r = pos_v.at[r]
            for j in range(D // L):
                sl = pl.ds(j * L, L)
                v = p_r[sl]
                o0_r[sl] = s0_r[sl] + v
                o1_r[sl] = s1_r[sl] + v
            return acc

        lax.fori_loop(0, H, row, 0)
        w_issue_h(c, 2 * ph, 0)
        w_issue_h(c, 2 * ph + 1, 0)
        lax.fori_loop(H, C, row, 0)
        w_issue_h(c, 2 * ph, 1)
        w_issue_h(c, 2 * ph + 1, 1)

    for b in range(B):
        pltpu.sync_copy(x_hbm.at[b, pl.ds(p0, P_PER_W)],
                        idx_v.at[pl.ds(b * P_PER_W, P_PER_W)])

    for b in range(B):
        g_issue(0, b)
    p_issue(0)
    p_wait()
    for ph in range(2):
        g_wait(2 * ph)
        g_wait(2 * ph + 1)
        if ph == 1:
            w_wait(0)
            w_wait(1)
        add_pair(0, ph)
        if ph == 1:
            p_issue(1)
        g_issue(1, 2 * ph)
        g_issue(1, 2 * ph + 1)

    def steady(c, acc):
        for ph in range(2):
            g_wait(2 * ph)
            g_wait(2 * ph + 1)
            if ph == 0:
                p_wait()
            w_wait(0)
            w_wait(1)
            add_pair(c, ph)
            if ph == 1:
                p_issue(c + 1)
            g_issue(c + 1, 2 * ph)
            g_issue(c + 1, 2 * ph + 1)
        return acc
    lax.fori_loop(1, NCH - 1, steady, 0)

    c = NCH - 1
    for ph in range(2):
        g_wait(2 * ph)
        g_wait(2 * ph + 1)
        if ph == 0:
            p_wait()
        w_wait(0)
        w_wait(1)
        add_pair(c, ph)
    w_wait(0)
    w_wait(1)


@jax.jit
def kernel(x, token_table, pos_table):
    mesh = plsc.VectorSubcoreMesh(
        core_axis_name="c", subcore_axis_name="s",
        num_cores=NC, num_subcores=NS)
    f = pl.kernel(
        _body,
        out_type=jax.ShapeDtypeStruct((B, T, D), jnp.float32),
        mesh=mesh,
        scratch_types=[
            pltpu.VMEM((B * P_PER_W,), jnp.int32),
            pltpu.VMEM((C, D), jnp.float32),
        ] + [pltpu.VMEM((C, D), jnp.float32)] * 6
          + [pltpu.SemaphoreType.DMA] * 7,
    )
    return f(x.astype(jnp.int32), token_table, pos_table)


# speedup vs baseline: 1.0085x; 1.0085x over previous
import jax
import jax.numpy as jnp
from jax import lax
from jax.experimental import pallas as pl
from jax.experimental.pallas import tpu as pltpu
from jax.experimental.pallas import tpu_sc as plsc

B = 4
T = 8192
D = 1024
NC = 2
NS = 16
NW = NC * NS
P_PER_W = T // NW
C = 16
NCH = P_PER_W // C
L = 16


def _body(x_hbm, tok_hbm, pos_hbm, out_hbm, idx_v, pos_v,
          t0, t1, t2, t3, ob0, ob1,
          g0, g1, g2, g3, w0, w1, psem):
    tok = (t0, t1, t2, t3)
    ob = (ob0, ob1)
    gsem = (g0, g1, g2, g3)
    wsem = (w0, w1)
    cid = lax.axis_index("c")
    sid = lax.axis_index("s")
    wid = sid * NC + cid
    p0 = wid * P_PER_W

    def g_issue(c, b):
        pltpu.async_copy(
            tok_hbm.at[idx_v.at[pl.ds(b * P_PER_W + c * C, C)]],
            tok[b], gsem[b])

    def g_wait(b):
        pltpu.make_async_copy(
            tok_hbm.at[idx_v.at[pl.ds(0, C)]], tok[b], gsem[b]).wait()

    def w_issue(c, b):
        pltpu.async_copy(ob[b % 2], out_hbm.at[b, pl.ds(p0 + c * C, C)],
                         wsem[b % 2])

    def w_wait(k):
        pltpu.make_async_copy(ob[k], out_hbm.at[0, pl.ds(0, C)],
                              wsem[k]).wait()

    def p_issue(c):
        pltpu.async_copy(pos_hbm.at[pl.ds(p0 + c * C, C)], pos_v, psem)

    def p_wait():
        pltpu.make_async_copy(pos_hbm.at[pl.ds(0, C)], pos_v, psem).wait()

    def add_pair(ph):
        s0, s1 = tok[2 * ph], tok[2 * ph + 1]

        @pl.loop(0, C, unroll=2)
        def row(r):
            o0_r = ob0.at[r]
            o1_r = ob1.at[r]
            s0_r = s0.at[r]
            s1_r = s1.at[r]
            p_r = pos_v.at[r]
            for j in range(D // L):
                sl = pl.ds(j * L, L)
                v = p_r[sl]
                o0_r[sl] = s0_r[sl] + v
                o1_r[sl] = s1_r[sl] + v

    for b in range(B):
        pltpu.sync_copy(x_hbm.at[b, pl.ds(p0, P_PER_W)],
                        idx_v.at[pl.ds(b * P_PER_W, P_PER_W)])

    for b in range(B):
        g_issue(0, b)
    p_issue(0)
    p_wait()
    for ph in range(2):
        g_wait(2 * ph)
        g_wait(2 * ph + 1)
        if ph == 1:
            w_wait(0)
            w_wait(1)
        add_pair(ph)
        if ph == 1:
            p_issue(1)
        w_issue(0, 2 * ph)
        w_issue(0, 2 * ph + 1)
        g_issue(1, 2 * ph)
        g_issue(1, 2 * ph + 1)

    def steady(c, acc):
        for ph in range(2):
            g_wait(2 * ph)
            g_wait(2 * ph + 1)
            if ph == 0:
                p_wait()
            w_wait(0)
            w_wait(1)
            add_pair(ph)
            if ph == 1:
                p_issue(c + 1)
            w_issue(c, 2 * ph)
            w_issue(c, 2 * ph + 1)
            g_issue(c + 1, 2 * ph)
            g_issue(c + 1, 2 * ph + 1)
        return acc
    lax.fori_loop(1, NCH - 1, steady, 0)

    c = NCH - 1
    for ph in range(2):
        g_wait(2 * ph)
        g_wait(2 * ph + 1)
        if ph == 0:
            p_wait()
        w_wait(0)
        w_wait(1)
        add_pair(ph)
        w_issue(c, 2 * ph)
        w_issue(c, 2 * ph + 1)
    w_wait(0)
    w_wait(1)


@jax.jit
def kernel(x, token_table, pos_table):
    mesh = plsc.VectorSubcoreMesh(
        core_axis_name="c", subcore_axis_name="s",
        num_cores=NC, num_subcores=NS)
    f = pl.kernel(
        _body,
        out_type=jax.ShapeDtypeStruct((B, T, D), jnp.float32),
        mesh=mesh,
        scratch_types=[
            pltpu.VMEM((B * P_PER_W,), jnp.int32),
            pltpu.VMEM((C, D), jnp.float32),
        ] + [pltpu.VMEM((C, D), jnp.float32)] * 6
          + [pltpu.SemaphoreType.DMA] * 7,
    )
    return f(x.astype(jnp.int32), token_table, pos_table)


# speedup vs baseline: 1.0652x; 1.0563x over previous
import jax
import jax.numpy as jnp
from jax import lax
from jax.experimental import pallas as pl
from jax.experimental.pallas import tpu as pltpu
from jax.experimental.pallas import tpu_sc as plsc

B = 4
T = 8192
D = 1024
NC = 2
NS = 16
NW = NC * NS
P_PER_W = T // NW
C = 16
NCH = P_PER_W // C
L = 16


def _body(x_hbm, tok_hbm, pos_hbm, out_hbm, idx_v, pos_v,
          t0, t1, t2, t3, ob0, ob1,
          g0, g1, g2, g3, w0, w1, psem):
    tok = (t0, t1, t2, t3)
    ob = (ob0, ob1)
    gsem = (g0, g1, g2, g3)
    wsem = (w0, w1)
    cid = lax.axis_index("c")
    sid = lax.axis_index("s")
    wid = sid * NC + cid
    p0 = wid * P_PER_W

    def g_issue(c, b):
        pltpu.async_copy(
            tok_hbm.at[idx_v.at[pl.ds(b * P_PER_W + c * C, C)]],
            tok[b], gsem[b])

    def g_wait(b):
        pltpu.make_async_copy(
            tok_hbm.at[idx_v.at[pl.ds(0, C)]], tok[b], gsem[b]).wait()

    def w_issue(c, b):
        pltpu.async_copy(ob[b % 2], out_hbm.at[b, pl.ds(p0 + c * C, C)],
                         wsem[b % 2])

    def w_wait(k):
        pltpu.make_async_copy(ob[k], out_hbm.at[0, pl.ds(0, C)],
                              wsem[k]).wait()

    def p_issue(c):
        pltpu.async_copy(pos_hbm.at[pl.ds(p0 + c * C, C)], pos_v, psem)

    def p_wait():
        pltpu.make_async_copy(pos_hbm.at[pl.ds(0, C)], pos_v, psem).wait()

    def add_pair(ph):
        s0, s1 = tok[2 * ph], tok[2 * ph + 1]

        @pl.loop(0, C)
        def row(r):
            o0_r = ob0.at[r]
            o1_r = ob1.at[r]
            s0_r = s0.at[r]
            s1_r = s1.at[r]
            p_r = pos_v.at[r]
            for j in range(D // L):
                sl = pl.ds(j * L, L)
                v = p_r[sl]
                o0_r[sl] = s0_r[sl] + v
                o1_r[sl] = s1_r[sl] + v

    for b in range(B):
        pltpu.sync_copy(x_hbm.at[b, pl.ds(p0, P_PER_W)],
                        idx_v.at[pl.ds(b * P_PER_W, P_PER_W)])

    for b in range(B):
        g_issue(0, b)
    p_issue(0)
    p_wait()
    for ph in range(2):
        g_wait(2 * ph)
        g_wait(2 * ph + 1)
        if ph == 1:
            w_wait(0)
            w_wait(1)
        add_pair(ph)
        if ph == 1:
            p_issue(1)
        w_issue(0, 2 * ph)
        w_issue(0, 2 * ph + 1)
        g_issue(1, 2 * ph)
        g_issue(1, 2 * ph + 1)

    def steady(c, acc):
        for ph in range(2):
            g_wait(2 * ph)
            g_wait(2 * ph + 1)
            if ph == 0:
                p_wait()
            w_wait(0)
            w_wait(1)
            add_pair(ph)
            if ph == 1:
                p_issue(c + 1)
            w_issue(c, 2 * ph)
            w_issue(c, 2 * ph + 1)
            g_issue(c + 1, 2 * ph)
            g_issue(c + 1, 2 * ph + 1)
        return acc
    lax.fori_loop(1, NCH - 1, steady, 0)

    c = NCH - 1
    for ph in range(2):
        g_wait(2 * ph)
        g_wait(2 * ph + 1)
        if ph == 0:
            p_wait()
        w_wait(0)
        w_wait(1)
        add_pair(ph)
        w_issue(c, 2 * ph)
        w_issue(c, 2 * ph + 1)
    w_wait(0)
    w_wait(1)


@jax.jit
def kernel(x, token_table, pos_table):
    mesh = plsc.VectorSubcoreMesh(
        core_axis_name="c", subcore_axis_name="s",
        num_cores=NC, num_subcores=NS)
    f = pl.kernel(
        _body,
        out_type=jax.ShapeDtypeStruct((B, T, D), jnp.float32),
        mesh=mesh,
        scratch_types=[
            pltpu.VMEM((B * P_PER_W,), jnp.int32),
            pltpu.VMEM((C, D), jnp.float32),
        ] + [pltpu.VMEM((C, D), jnp.float32)] * 6
          + [pltpu.SemaphoreType.DMA] * 7,
    )
    return f(x.astype(jnp.int32), token_table, pos_table)


# speedup vs baseline: 1.0745x; 1.0087x over previous
import jax
import jax.numpy as jnp
from jax import lax
from jax.experimental import pallas as pl
from jax.experimental.pallas import tpu as pltpu
from jax.experimental.pallas import tpu_sc as plsc

B = 4
T = 8192
D = 1024
NC = 2
NS = 16
NW = NC * NS
P_PER_W = T // NW
C = 16
NCH = P_PER_W // C
L = 16


def _body(x_hbm, tok_hbm, pos_hbm, out_hbm, idx_v, pos_v,
          t0, t1, t2, t3, ob0, ob1,
          g0, g1, g2, g3, w0, w1, psem):
    tok = (t0, t1, t2, t3)
    ob = (ob0, ob1)
    gsem = (g0, g1, g2, g3)
    wsem = (w0, w1)
    cid = lax.axis_index("c")
    sid = lax.axis_index("s")
    wid = sid * NC + cid
    p0 = wid * P_PER_W

    def g_issue(c, b):
        pltpu.async_copy(
            tok_hbm.at[idx_v.at[b, pl.ds(c * C, C)]],
            tok[b], gsem[b])

    def g_wait(b):
        pltpu.make_async_copy(
            tok_hbm.at[idx_v.at[0, pl.ds(0, C)]], tok[b], gsem[b]).wait()

    def w_issue(c, b):
        pltpu.async_copy(ob[b % 2], out_hbm.at[b, pl.ds(p0 + c * C, C)],
                         wsem[b % 2])

    def w_wait(k):
        pltpu.make_async_copy(ob[k], out_hbm.at[0, pl.ds(0, C)],
                              wsem[k]).wait()

    def p_issue(c):
        pltpu.async_copy(pos_hbm.at[pl.ds(p0 + c * C, C)], pos_v, psem)

    def p_wait():
        pltpu.make_async_copy(pos_hbm.at[pl.ds(0, C)], pos_v, psem).wait()

    def add_pair(ph):
        s0, s1 = tok[2 * ph], tok[2 * ph + 1]

        @pl.loop(0, C)
        def row(r):
            o0_r = ob0.at[r]
            o1_r = ob1.at[r]
            s0_r = s0.at[r]
            s1_r = s1.at[r]
            p_r = pos_v.at[r]
            for j in range(D // L):
                sl = pl.ds(j * L, L)
                v = p_r[sl]
                o0_r[sl] = s0_r[sl] + v
                o1_r[sl] = s1_r[sl] + v

    pltpu.sync_copy(x_hbm.at[:, pl.ds(p0, P_PER_W)], idx_v)

    for b in range(B):
        g_issue(0, b)
    p_issue(0)
    p_wait()
    for ph in range(2):
        g_wait(2 * ph)
        g_wait(2 * ph + 1)
        if ph == 1:
            w_wait(0)
            w_wait(1)
        add_pair(ph)
        if ph == 1:
            p_issue(1)
        w_issue(0, 2 * ph)
        w_issue(0, 2 * ph + 1)
        g_issue(1, 2 * ph)
        g_issue(1, 2 * ph + 1)

    def steady(c, acc):
        for ph in range(2):
            g_wait(2 * ph)
            g_wait(2 * ph + 1)
            if ph == 0:
                p_wait()
            w_wait(0)
            w_wait(1)
            add_pair(ph)
            if ph == 1:
                p_issue(c + 1)
            w_issue(c, 2 * ph)
            w_issue(c, 2 * ph + 1)
            g_issue(c + 1, 2 * ph)
            g_issue(c + 1, 2 * ph + 1)
        return acc
    lax.fori_loop(1, NCH - 1, steady, 0)

    c = NCH - 1
    for ph in range(2):
        g_wait(2 * ph)
        g_wait(2 * ph + 1)
        if ph == 0:
            p_wait()
        w_wait(0)
        w_wait(1)
        add_pair(ph)
        w_issue(c, 2 * ph)
        w_issue(c, 2 * ph + 1)
    w_wait(0)
    w_wait(1)


@jax.jit
def kernel(x, token_table, pos_table):
    mesh = plsc.VectorSubcoreMesh(
        core_axis_name="c", subcore_axis_name="s",
        num_cores=NC, num_subcores=NS)
    f = pl.kernel(
        _body,
        out_type=jax.ShapeDtypeStruct((B, T, D), jnp.float32),
        mesh=mesh,
        scratch_types=[
            pltpu.VMEM((B, P_PER_W), jnp.int32),
            pltpu.VMEM((C, D), jnp.float32),
        ] + [pltpu.VMEM((C, D), jnp.float32)] * 6
          + [pltpu.SemaphoreType.DMA] * 7,
    )
    return f(x.astype(jnp.int32), token_table, pos_table)
